# Initial kernel scaffold; baseline (speedup 1.0000x reference)
import jax
import jax.numpy as jnp
from jax.experimental import pallas as pl


def kernel(state, mode, x, edge_index, g1_w1, g1_b1, g1_w2, g1_b2, g2_w1, g2_b1, g2_w2, g2_b2, mode_emb, fc1_w, fc1_b, fc2_w, fc2_b, mean_w, mean_b, ls_w, ls_b):
    raise NotImplementedError("write your pallas kernel here")


# speedup vs baseline: 3.8652x; 3.8652x over previous
import functools

import jax
import jax.numpy as jnp
from jax import lax
from jax.experimental import pallas as pl
from jax.experimental.pallas import tpu as pltpu
from jax.experimental.pallas import tpu_sc as plsc

N = 10000
E = 320000
D = 128
GNN = 64
H = 256
AOUT = 32
MODES = 3
MEDIM = 16

NC = 2
NS = 16
NW = NC * NS
LANES = 16

C = 128
EPW = 79 * C
E_PAD = EPW * NW
N_PAD = 10240
ZR = N_PAD // NS
RB = 256
GRID = N_PAD // RB

_SC_MESH = None


def _sc_mesh():
    global _SC_MESH
    if _SC_MESH is None:
        _SC_MESH = plsc.VectorSubcoreMesh(core_axis_name="c", subcore_axis_name="s")
    return _SC_MESH


def _tc_ab_body(x_ref, wa_ref, wb_ref, b1_ref, a_ref, b_ref):
    xb = x_ref[...]
    a_ref[...] = (
        jnp.dot(xb, wa_ref[...], preferred_element_type=jnp.float32) + b1_ref[...]
    )
    b_ref[...] = jnp.dot(xb, wb_ref[...], preferred_element_type=jnp.float32)


def _tc_ab(xp, wa, wb, b1):
    F = xp.shape[1]
    K = wa.shape[1]
    return pl.pallas_call(
        _tc_ab_body,
        grid=(GRID,),
        in_specs=[
            pl.BlockSpec((RB, F), lambda i: (i, 0)),
            pl.BlockSpec((F, K), lambda i: (0, 0)),
            pl.BlockSpec((F, K), lambda i: (0, 0)),
            pl.BlockSpec((1, K), lambda i: (0, 0)),
        ],
        out_specs=[
            pl.BlockSpec((RB, K), lambda i: (i, 0)),
            pl.BlockSpec((RB, K), lambda i: (i, 0)),
        ],
        out_shape=[
            jax.ShapeDtypeStruct((N_PAD, K), jnp.float32),
            jax.ShapeDtypeStruct((N_PAD, K), jnp.float32),
        ],
    )(xp, wa, wb, b1)


def _tc_mid_body(acc_ref, cnt_ref, w2_ref, b2_ref, wa_ref, wb_ref, b1_ref,
                 a2_ref, b2out_ref):
    S = acc_ref[0] + acc_ref[1]
    c = (cnt_ref[0, 0, :] + cnt_ref[1, 0, :]).reshape(RB, 1)
    mean = S / jnp.maximum(c, 1.0)
    o = jnp.dot(mean, w2_ref[...], preferred_element_type=jnp.float32) + b2_ref[...]
    h = jnp.maximum(o * (c > 0.0).astype(jnp.float32), 0.0)
    a2_ref[...] = (
        jnp.dot(h, wa_ref[...], preferred_element_type=jnp.float32) + b1_ref[...]
    )
    b2out_ref[...] = jnp.dot(h, wb_ref[...], preferred_element_type=jnp.float32)


def _tc_mid(acc1, cnt, w2, b2, wa, wb, b1):
    K = acc1.shape[2]
    G = wa.shape[1]
    return pl.pallas_call(
        _tc_mid_body,
        grid=(GRID,),
        in_specs=[
            pl.BlockSpec((2, RB, K), lambda i: (0, i, 0)),
            pl.BlockSpec((2, 1, RB), lambda i: (0, 0, i)),
            pl.BlockSpec((K, K), lambda i: (0, 0)),
            pl.BlockSpec((1, K), lambda i: (0, 0)),
            pl.BlockSpec((K, G), lambda i: (0, 0)),
            pl.BlockSpec((K, G), lambda i: (0, 0)),
            pl.BlockSpec((1, G), lambda i: (0, 0)),
        ],
        out_specs=[
            pl.BlockSpec((RB, G), lambda i: (i, 0)),
            pl.BlockSpec((RB, G), lambda i: (i, 0)),
        ],
        out_shape=[
            jax.ShapeDtypeStruct((N_PAD, G), jnp.float32),
            jax.ShapeDtypeStruct((N_PAD, G), jnp.float32),
        ],
    )(acc1, cnt, w2, b2, wa, wb, b1)


def _tc_head_body(acc_ref, cnt_ref, w2_ref, b2_ref, state_ref, mode_ref, memb_ref,
                  f1s_ref, f1g_ref, f1m_ref, f1b_ref, f2w_ref, f2b_ref,
                  mw_ref, mb_ref, lw_ref, lb_ref, mean_ref, ls_ref):
    S = acc_ref[0] + acc_ref[1]
    c = (cnt_ref[0, 0, :] + cnt_ref[1, 0, :]).reshape(RB, 1)
    meanagg = S / jnp.maximum(c, 1.0)
    g = (
        jnp.dot(meanagg, w2_ref[...], preferred_element_type=jnp.float32)
        + b2_ref[...]
    ) * (c > 0.0).astype(jnp.float32)
    oh = (mode_ref[...] == lax.broadcasted_iota(jnp.int32, (1, 8), 1)).astype(
        jnp.float32
    )
    me = jnp.dot(oh, memb_ref[...], preferred_element_type=jnp.float32)
    z = jnp.maximum(
        jnp.dot(state_ref[...], f1s_ref[...], preferred_element_type=jnp.float32)
        + jnp.dot(g, f1g_ref[...], preferred_element_type=jnp.float32)
        + jnp.dot(me, f1m_ref[...], preferred_element_type=jnp.float32)
        + f1b_ref[...],
        0.0,
    )
    z = jnp.maximum(
        jnp.dot(z, f2w_ref[...], preferred_element_type=jnp.float32) + f2b_ref[...],
        0.0,
    )
    mean_ref[...] = (
        jnp.dot(z, mw_ref[...], preferred_element_type=jnp.float32) + mb_ref[...]
    )
    ls_ref[...] = jnp.clip(
        jnp.dot(z, lw_ref[...], preferred_element_type=jnp.float32) + lb_ref[...],
        -20.0,
        2.0,
    )


def _tc_head(acc2, cnt, w2, b2, statep, modep, memb, f1s, f1g, f1m, f1b,
             f2w, f2b, mw, mb, lw, lb):
    G = acc2.shape[2]
    return pl.pallas_call(
        _tc_head_body,
        grid=(GRID,),
        in_specs=[
            pl.BlockSpec((2, RB, G), lambda i: (0, i, 0)),
            pl.BlockSpec((2, 1, RB), lambda i: (0, 0, i)),
            pl.BlockSpec((G, GNN), lambda i: (0, 0)),
            pl.BlockSpec((1, GNN), lambda i: (0, 0)),
            pl.BlockSpec((RB, D), lambda i: (i, 0)),
            pl.BlockSpec((RB, 1), lambda i: (i, 0)),
            pl.BlockSpec((8, MEDIM), lambda i: (0, 0)),
            pl.BlockSpec((D, H), lambda i: (0, 0)),
            pl.BlockSpec((GNN, H), lambda i: (0, 0)),
            pl.BlockSpec((MEDIM, H), lambda i: (0, 0)),
            pl.BlockSpec((1, H), lambda i: (0, 0)),
            pl.BlockSpec((H, H), lambda i: (0, 0)),
            pl.BlockSpec((1, H), lambda i: (0, 0)),
            pl.BlockSpec((H, AOUT), lambda i: (0, 0)),
            pl.BlockSpec((1, AOUT), lambda i: (0, 0)),
            pl.BlockSpec((H, AOUT), lambda i: (0, 0)),
            pl.BlockSpec((1, AOUT), lambda i: (0, 0)),
        ],
        out_specs=[
            pl.BlockSpec((RB, AOUT), lambda i: (i, 0)),
            pl.BlockSpec((RB, AOUT), lambda i: (i, 0)),
        ],
        out_shape=[
            jax.ShapeDtypeStruct((N_PAD, AOUT), jnp.float32),
            jax.ShapeDtypeStruct((N_PAD, AOUT), jnp.float32),
        ],
    )(acc2, cnt, w2, b2, statep, modep, memb, f1s, f1g, f1m, f1b,
      f2w, f2b, mw, mb, lw, lb)


def _sc_edge_call(a_hbm_arr, b_hbm_arr, dst_arr, src_arr, K):
    out_type = [jax.ShapeDtypeStruct((NC, N_PAD, K), jnp.float32)]
    scratch = [
        pltpu.VMEM((1, C), jnp.int32),
        pltpu.VMEM((C,), jnp.int32),
        pltpu.VMEM((C, K), jnp.float32),
        pltpu.VMEM((C, K), jnp.float32),
        pltpu.SemaphoreType.DMA,
        pltpu.SemaphoreType.DMA,
        pltpu.VMEM_SHARED((N_PAD, K), jnp.float32),
    ]

    def body(a_hbm, b_hbm, dst_hbm, src_hbm, acc_out,
             dbuf, sbuf, arows, brows, sem_a, sem_b, acc_sh):
        cid = lax.axis_index("c")
        sid = lax.axis_index("s")
        wid = cid * NS + sid

        @pl.loop(0, C)
        def _zero_stage(e):
            for j in range(K // LANES):
                arows[e, pl.ds(j * LANES, LANES)] = jnp.zeros((LANES,), jnp.float32)

        @pl.loop(0, ZR // C)
        def _zero_acc(z):
            r0 = sid * ZR + z * C
            pltpu.sync_copy(arows, acc_sh.at[pl.ds(r0, C)])

        plsc.subcore_barrier()

        ebase = wid * EPW

        @pl.loop(0, EPW // C)
        def _chunk(ci):
            e0 = ebase + ci * C
            pltpu.sync_copy(dst_hbm.at[pl.ds(e0, C)], dbuf.at[0])
            pltpu.sync_copy(src_hbm.at[pl.ds(e0, C)], sbuf)
            cpa = pltpu.async_copy(a_hbm.at[dbuf.at[0]], arows, sem_a)
            cpb = pltpu.async_copy(b_hbm.at[sbuf], brows, sem_b)
            cpa.wait()
            cpb.wait()

            @pl.loop(0, C)
            def _compute(e):
                for j in range(K // LANES):
                    sl = pl.ds(j * LANES, LANES)
                    arows[e, sl] = jnp.maximum(arows[e, sl] + brows[e, sl], 0.0)

            pltpu.sync_copy(arows, acc_sh.at[dbuf.at[0]], add=True)

        plsc.subcore_barrier()

        @pl.loop(0, ZR // C)
        def _copy_out(z):
            r0 = sid * ZR + z * C
            pltpu.sync_copy(acc_sh.at[pl.ds(r0, C)], acc_out.at[cid, pl.ds(r0, C)])

    fn = pl.kernel(body, out_type=out_type, mesh=_sc_mesh(), scratch_types=scratch)
    return fn(a_hbm_arr, b_hbm_arr, dst_arr, src_arr)


def _sc_cnt_call(dst_arr):
    out_type = [jax.ShapeDtypeStruct((NC, N_PAD), jnp.float32)]
    scratch = [
        pltpu.VMEM((1, C), jnp.int32),
        pltpu.VMEM((C,), jnp.float32),
        pltpu.VMEM_SHARED((N_PAD,), jnp.float32),
    ]

    def body(dst_hbm, cnt_out, dbuf, ones, cnt_sh):
        cid = lax.axis_index("c")
        sid = lax.axis_index("s")
        wid = cid * NS + sid

        @pl.loop(0, C, step=LANES)
        def _zero_stage(e):
            ones[pl.ds(e, LANES)] = jnp.zeros((LANES,), jnp.float32)

        @pl.loop(0, ZR // C)
        def _zero_acc(z):
            pltpu.sync_copy(ones, cnt_sh.at[pl.ds(sid * ZR + z * C, C)])

        @pl.loop(0, C, step=LANES)
        def _fill_ones(e):
            ones[pl.ds(e, LANES)] = jnp.ones((LANES,), jnp.float32)

        plsc.subcore_barrier()

        ebase = wid * EPW

        @pl.loop(0, EPW // C)
        def _chunk(ci):
            e0 = ebase + ci * C
            pltpu.sync_copy(dst_hbm.at[pl.ds(e0, C)], dbuf.at[0])
            pltpu.sync_copy(ones, cnt_sh.at[dbuf.at[0]], add=True)

        plsc.subcore_barrier()

        pltpu.sync_copy(
            cnt_sh.at[pl.ds(sid * ZR, ZR)], cnt_out.at[cid, pl.ds(sid * ZR, ZR)]
        )

    fn = pl.kernel(body, out_type=out_type, mesh=_sc_mesh(), scratch_types=scratch)
    return fn(dst_arr)


def kernel(state, mode, x, edge_index, g1_w1, g1_b1, g1_w2, g1_b2,
           g2_w1, g2_b1, g2_w2, g2_b2, mode_emb,
           fc1_w, fc1_b, fc2_w, fc2_b, mean_w, mean_b, ls_w, ls_b):
    f32 = jnp.float32
    xp = jnp.zeros((N_PAD, D), f32).at[:N].set(x)
    statep = jnp.zeros((N_PAD, D), f32).at[:N].set(state)
    modep = jnp.zeros((N_PAD, 1), jnp.int32).at[:N, 0].set(mode)
    membp = jnp.zeros((8, MEDIM), f32).at[:MODES].set(mode_emb)
    pad = jnp.full((E_PAD - E,), N, jnp.int32)
    dstp = jnp.concatenate([edge_index[1], pad])
    srcp = jnp.concatenate([edge_index[0], pad])

    (cnt,) = _sc_cnt_call(dstp)
    cnt = cnt.reshape(NC, 1, N_PAD)
    a1, b1arr = _tc_ab(xp, g1_w1[:D], g1_w1[D:], g1_b1.reshape(1, -1))
    (acc1,) = _sc_edge_call(a1, b1arr, dstp, srcp, 128)

    w2a_p = jnp.zeros((128, 128), f32).at[:, :GNN].set(g2_w1[:128])
    w2b_p = jnp.zeros((128, 128), f32).at[:, :GNN].set(g2_w1[128:])
    b21_p = jnp.zeros((1, 128), f32).at[0, :GNN].set(g2_b1)
    a2, b2arr = _tc_mid(acc1, cnt, g1_w2, g1_b2.reshape(1, -1),
                        w2a_p, w2b_p, b21_p)
    (acc2,) = _sc_edge_call(a2, b2arr, dstp, srcp, 128)

    g2w2_p = jnp.zeros((128, GNN), f32).at[:GNN].set(g2_w2)
    meanp, lsp = _tc_head(
        acc2, cnt, g2w2_p, g2_b2.reshape(1, -1), statep, modep, membp,
        fc1_w[:D], fc1_w[D:D + GNN], fc1_w[D + GNN:], fc1_b.reshape(1, -1),
        fc2_w, fc2_b.reshape(1, -1), mean_w, mean_b.reshape(1, -1),
        ls_w, ls_b.reshape(1, -1),
    )
    return meanp[:N], lsp[:N]


# speedup vs baseline: 3.9801x; 1.0297x over previous
import functools

import jax
import jax.numpy as jnp
from jax import lax
from jax.experimental import pallas as pl
from jax.experimental.pallas import tpu as pltpu
from jax.experimental.pallas import tpu_sc as plsc

N = 10000
E = 320000
D = 128
GNN = 64
H = 256
AOUT = 32
MODES = 3
MEDIM = 16

NC = 2
NS = 16
NW = NC * NS
LANES = 16

C = 64
NCH = 160
EPW = NCH * C
E_PAD = EPW * NW
NCHTOT = E_PAD // C
N_PAD = 10240
ZR = N_PAD // NS
RB = 256
GRID = N_PAD // RB

_SC_MESH = None


def _sc_mesh():
    global _SC_MESH
    if _SC_MESH is None:
        _SC_MESH = plsc.VectorSubcoreMesh(core_axis_name="c", subcore_axis_name="s")
    return _SC_MESH


def _tc_ab_body(x_ref, wa_ref, wb_ref, b1_ref, a_ref, b_ref):
    xb = x_ref[...]
    a_ref[...] = (
        jnp.dot(xb, wa_ref[...], preferred_element_type=jnp.float32) + b1_ref[...]
    )
    b_ref[...] = jnp.dot(xb, wb_ref[...], preferred_element_type=jnp.float32)


def _tc_ab(xp, wa, wb, b1):
    F = xp.shape[1]
    K = wa.shape[1]
    return pl.pallas_call(
        _tc_ab_body,
        grid=(GRID,),
        in_specs=[
            pl.BlockSpec((RB, F), lambda i: (i, 0)),
            pl.BlockSpec((F, K), lambda i: (0, 0)),
            pl.BlockSpec((F, K), lambda i: (0, 0)),
            pl.BlockSpec((1, K), lambda i: (0, 0)),
        ],
        out_specs=[
            pl.BlockSpec((RB, K), lambda i: (i, 0)),
            pl.BlockSpec((RB, K), lambda i: (i, 0)),
        ],
        out_shape=[
            jax.ShapeDtypeStruct((N_PAD, K), jnp.float32),
            jax.ShapeDtypeStruct((N_PAD, K), jnp.float32),
        ],
    )(xp, wa, wb, b1)


def _tc_mid_body(acc_ref, cnt_ref, w2_ref, b2_ref, wa_ref, wb_ref, b1_ref,
                 a2_ref, b2out_ref):
    S = acc_ref[0] + acc_ref[1]
    c = (cnt_ref[0, 0, :] + cnt_ref[1, 0, :]).reshape(RB, 1)
    mean = S / jnp.maximum(c, 1.0)
    o = jnp.dot(mean, w2_ref[...], preferred_element_type=jnp.float32) + b2_ref[...]
    h = jnp.maximum(o * (c > 0.0).astype(jnp.float32), 0.0)
    a2_ref[...] = (
        jnp.dot(h, wa_ref[...], preferred_element_type=jnp.float32) + b1_ref[...]
    )
    b2out_ref[...] = jnp.dot(h, wb_ref[...], preferred_element_type=jnp.float32)


def _tc_mid(acc1, cnt, w2, b2, wa, wb, b1):
    K = acc1.shape[2]
    G = wa.shape[1]
    return pl.pallas_call(
        _tc_mid_body,
        grid=(GRID,),
        in_specs=[
            pl.BlockSpec((2, RB, K), lambda i: (0, i, 0)),
            pl.BlockSpec((2, 1, RB), lambda i: (0, 0, i)),
            pl.BlockSpec((K, K), lambda i: (0, 0)),
            pl.BlockSpec((1, K), lambda i: (0, 0)),
            pl.BlockSpec((K, G), lambda i: (0, 0)),
            pl.BlockSpec((K, G), lambda i: (0, 0)),
            pl.BlockSpec((1, G), lambda i: (0, 0)),
        ],
        out_specs=[
            pl.BlockSpec((RB, G), lambda i: (i, 0)),
            pl.BlockSpec((RB, G), lambda i: (i, 0)),
        ],
        out_shape=[
            jax.ShapeDtypeStruct((N_PAD, G), jnp.float32),
            jax.ShapeDtypeStruct((N_PAD, G), jnp.float32),
        ],
    )(acc1, cnt, w2, b2, wa, wb, b1)


def _tc_head_body(acc_ref, cnt_ref, w2_ref, b2_ref, state_ref, mode_ref, memb_ref,
                  f1s_ref, f1g_ref, f1m_ref, f1b_ref, f2w_ref, f2b_ref,
                  mw_ref, mb_ref, lw_ref, lb_ref, mean_ref, ls_ref):
    S = acc_ref[0] + acc_ref[1]
    c = (cnt_ref[0, 0, :] + cnt_ref[1, 0, :]).reshape(RB, 1)
    meanagg = S / jnp.maximum(c, 1.0)
    g = (
        jnp.dot(meanagg, w2_ref[...], preferred_element_type=jnp.float32)
        + b2_ref[...]
    ) * (c > 0.0).astype(jnp.float32)
    oh = (mode_ref[...] == lax.broadcasted_iota(jnp.int32, (1, 8), 1)).astype(
        jnp.float32
    )
    me = jnp.dot(oh, memb_ref[...], preferred_element_type=jnp.float32)
    z = jnp.maximum(
        jnp.dot(state_ref[...], f1s_ref[...], preferred_element_type=jnp.float32)
        + jnp.dot(g, f1g_ref[...], preferred_element_type=jnp.float32)
        + jnp.dot(me, f1m_ref[...], preferred_element_type=jnp.float32)
        + f1b_ref[...],
        0.0,
    )
    z = jnp.maximum(
        jnp.dot(z, f2w_ref[...], preferred_element_type=jnp.float32) + f2b_ref[...],
        0.0,
    )
    mean_ref[...] = (
        jnp.dot(z, mw_ref[...], preferred_element_type=jnp.float32) + mb_ref[...]
    )
    ls_ref[...] = jnp.clip(
        jnp.dot(z, lw_ref[...], preferred_element_type=jnp.float32) + lb_ref[...],
        -20.0,
        2.0,
    )


def _tc_head(acc2, cnt, w2, b2, statep, modep, memb, f1s, f1g, f1m, f1b,
             f2w, f2b, mw, mb, lw, lb):
    G = acc2.shape[2]
    return pl.pallas_call(
        _tc_head_body,
        grid=(GRID,),
        in_specs=[
            pl.BlockSpec((2, RB, G), lambda i: (0, i, 0)),
            pl.BlockSpec((2, 1, RB), lambda i: (0, 0, i)),
            pl.BlockSpec((G, GNN), lambda i: (0, 0)),
            pl.BlockSpec((1, GNN), lambda i: (0, 0)),
            pl.BlockSpec((RB, D), lambda i: (i, 0)),
            pl.BlockSpec((RB, 1), lambda i: (i, 0)),
            pl.BlockSpec((8, MEDIM), lambda i: (0, 0)),
            pl.BlockSpec((D, H), lambda i: (0, 0)),
            pl.BlockSpec((GNN, H), lambda i: (0, 0)),
            pl.BlockSpec((MEDIM, H), lambda i: (0, 0)),
            pl.BlockSpec((1, H), lambda i: (0, 0)),
            pl.BlockSpec((H, H), lambda i: (0, 0)),
            pl.BlockSpec((1, H), lambda i: (0, 0)),
            pl.BlockSpec((H, AOUT), lambda i: (0, 0)),
            pl.BlockSpec((1, AOUT), lambda i: (0, 0)),
            pl.BlockSpec((H, AOUT), lambda i: (0, 0)),
            pl.BlockSpec((1, AOUT), lambda i: (0, 0)),
        ],
        out_specs=[
            pl.BlockSpec((RB, AOUT), lambda i: (i, 0)),
            pl.BlockSpec((RB, AOUT), lambda i: (i, 0)),
        ],
        out_shape=[
            jax.ShapeDtypeStruct((N_PAD, AOUT), jnp.float32),
            jax.ShapeDtypeStruct((N_PAD, AOUT), jnp.float32),
        ],
    )(acc2, cnt, w2, b2, statep, modep, memb, f1s, f1g, f1m, f1b,
      f2w, f2b, mw, mb, lw, lb)


def _sc_edge_call(a_hbm_arr, b_hbm_arr, ip_arr, K):
    out_type = [jax.ShapeDtypeStruct((NC, N_PAD, K), jnp.float32)]
    scratch = [
        pltpu.VMEM((4, 2, C), jnp.int32),
        pltpu.VMEM((2, C, K), jnp.float32),
        pltpu.VMEM((2, C, K), jnp.float32),
    ] + [pltpu.SemaphoreType.DMA] * 10 + [
        pltpu.VMEM_SHARED((N_PAD, K), jnp.float32),
    ]

    def body(a_hbm, b_hbm, ip_hbm, acc_out,
             ibuf, arows, brows,
             si0, si1, si2, si3, sga0, sga1, sgb0, sgb1, ssc0, ssc1,
             acc_sh):
        cid = lax.axis_index("c")
        sid = lax.axis_index("s")
        wid = cid * NS + sid
        si = (si0, si1, si2, si3)
        sga = (sga0, sga1)
        sgb = (sgb0, sgb1)
        ssc = (ssc0, ssc1)
        gbase = wid * NCH

        def issue_idx(cc, q):
            pltpu.async_copy(ip_hbm.at[gbase + cc], ibuf.at[q], si[q])

        def wait_idx(cc, q):
            pltpu.make_async_copy(ip_hbm.at[gbase + cc], ibuf.at[q], si[q]).wait()

        def issue_gathers(b, q):
            pltpu.async_copy(a_hbm.at[ibuf.at[q, 0]], arows.at[b], sga[b])
            pltpu.async_copy(b_hbm.at[ibuf.at[q, 1]], brows.at[b], sgb[b])

        def wait_gathers(b, q):
            pltpu.make_async_copy(a_hbm.at[ibuf.at[q, 0]], arows.at[b], sga[b]).wait()
            pltpu.make_async_copy(b_hbm.at[ibuf.at[q, 1]], brows.at[b], sgb[b]).wait()

        def issue_scatter(b, q):
            pltpu.async_copy(arows.at[b], acc_sh.at[ibuf.at[q, 0]], ssc[b], add=True)

        def wait_scatter(b, q):
            pltpu.make_async_copy(arows.at[b], acc_sh.at[ibuf.at[q, 0]], ssc[b]).wait()

        def compute(b):
            @pl.loop(0, C, step=2)
            def _cp(e):
                for ee in range(2):
                    for j in range(K // LANES):
                        sl = pl.ds(j * LANES, LANES)
                        arows[b, e + ee, sl] = jnp.maximum(
                            arows[b, e + ee, sl] + brows[b, e + ee, sl], 0.0
                        )

        @pl.loop(0, C)
        def _zero_stage(e):
            for j in range(K // LANES):
                arows[0, e, pl.ds(j * LANES, LANES)] = jnp.zeros(
                    (LANES,), jnp.float32
                )

        @pl.loop(0, ZR // C)
        def _zero_acc(z):
            pltpu.sync_copy(arows.at[0], acc_sh.at[pl.ds(sid * ZR + z * C, C)])

        plsc.subcore_barrier()

        issue_idx(0, 0)
        issue_idx(1, 1)
        issue_idx(2, 2)
        wait_idx(0, 0)
        issue_gathers(0, 0)

        @pl.loop(0, NCH, step=4)
        def _quad(ci):
            for s in range(4):
                cc = ci + s
                b = s % 2
                nb = (s + 1) % 2

                @pl.when(cc > 0)
                def _(cc=cc, nb=nb, s=s):
                    wait_scatter(nb, (s + 3) % 4)

                @pl.when(cc + 1 < NCH)
                def _(cc=cc, nb=nb, s=s):
                    wait_idx(cc + 1, (s + 1) % 4)
                    issue_gathers(nb, (s + 1) % 4)

                @pl.when(cc + 3 < NCH)
                def _(cc=cc, s=s):
                    issue_idx(cc + 3, (s + 3) % 4)

                wait_gathers(b, s % 4)
                compute(b)
                issue_scatter(b, s % 4)

        wait_scatter(1, 3)

        plsc.subcore_barrier()

        pltpu.sync_copy(
            acc_sh.at[pl.ds(sid * ZR, ZR)], acc_out.at[cid, pl.ds(sid * ZR, ZR)]
        )

    fn = pl.kernel(body, out_type=out_type, mesh=_sc_mesh(), scratch_types=scratch)
    return fn(a_hbm_arr, b_hbm_arr, ip_arr)


def _sc_cnt_call(dst_arr):
    out_type = [jax.ShapeDtypeStruct((NC, N_PAD), jnp.float32)]
    scratch = [
        pltpu.VMEM((1, C), jnp.int32),
        pltpu.VMEM((C,), jnp.float32),
        pltpu.VMEM_SHARED((N_PAD,), jnp.float32),
    ]

    def body(dst_hbm, cnt_out, dbuf, ones, cnt_sh):
        cid = lax.axis_index("c")
        sid = lax.axis_index("s")
        wid = cid * NS + sid

        @pl.loop(0, C, step=LANES)
        def _zero_stage(e):
            ones[pl.ds(e, LANES)] = jnp.zeros((LANES,), jnp.float32)

        @pl.loop(0, ZR // C)
        def _zero_acc(z):
            pltpu.sync_copy(ones, cnt_sh.at[pl.ds(sid * ZR + z * C, C)])

        @pl.loop(0, C, step=LANES)
        def _fill_ones(e):
            ones[pl.ds(e, LANES)] = jnp.ones((LANES,), jnp.float32)

        plsc.subcore_barrier()

        ebase = wid * EPW

        @pl.loop(0, EPW // C)
        def _chunk(ci):
            e0 = ebase + ci * C
            pltpu.sync_copy(dst_hbm.at[pl.ds(e0, C)], dbuf.at[0])
            pltpu.sync_copy(ones, cnt_sh.at[dbuf.at[0]], add=True)

        plsc.subcore_barrier()

        pltpu.sync_copy(
            cnt_sh.at[pl.ds(sid * ZR, ZR)], cnt_out.at[cid, pl.ds(sid * ZR, ZR)]
        )

    fn = pl.kernel(body, out_type=out_type, mesh=_sc_mesh(), scratch_types=scratch)
    return fn(dst_arr)


def kernel(state, mode, x, edge_index, g1_w1, g1_b1, g1_w2, g1_b2,
           g2_w1, g2_b1, g2_w2, g2_b2, mode_emb,
           fc1_w, fc1_b, fc2_w, fc2_b, mean_w, mean_b, ls_w, ls_b):
    f32 = jnp.float32
    xp = jnp.zeros((N_PAD, D), f32).at[:N].set(x)
    statep = jnp.zeros((N_PAD, D), f32).at[:N].set(state)
    modep = jnp.zeros((N_PAD, 1), jnp.int32).at[:N, 0].set(mode)
    membp = jnp.zeros((8, MEDIM), f32).at[:MODES].set(mode_emb)
    pad = jnp.full((E_PAD - E,), N, jnp.int32)
    dstp = jnp.concatenate([edge_index[1], pad])
    srcp = jnp.concatenate([edge_index[0], pad])
    ipacked = jnp.stack(
        [dstp.reshape(NCHTOT, C), srcp.reshape(NCHTOT, C)], axis=1
    )

    (cnt,) = _sc_cnt_call(dstp)
    cnt = cnt.reshape(NC, 1, N_PAD)
    a1, b1arr = _tc_ab(xp, g1_w1[:D], g1_w1[D:], g1_b1.reshape(1, -1))
    (acc1,) = _sc_edge_call(a1, b1arr, ipacked, 128)

    w2a_p = jnp.zeros((128, 128), f32).at[:, :GNN].set(g2_w1[:128])
    w2b_p = jnp.zeros((128, 128), f32).at[:, :GNN].set(g2_w1[128:])
    b21_p = jnp.zeros((1, 128), f32).at[0, :GNN].set(g2_b1)
    a2, b2arr = _tc_mid(acc1, cnt, g1_w2, g1_b2.reshape(1, -1),
                        w2a_p, w2b_p, b21_p)
    (acc2,) = _sc_edge_call(a2, b2arr, ipacked, 128)

    g2w2_p = jnp.zeros((128, GNN), f32).at[:GNN].set(g2_w2)
    meanp, lsp = _tc_head(
        acc2, cnt, g2w2_p, g2_b2.reshape(1, -1), statep, modep, membp,
        fc1_w[:D], fc1_w[D:D + GNN], fc1_w[D + GNN:], fc1_b.reshape(1, -1),
        fc2_w, fc2_b.reshape(1, -1), mean_w, mean_b.reshape(1, -1),
        ls_w, ls_b.reshape(1, -1),
    )
    return meanp[:N], lsp[:N]
